# Initial kernel scaffold; baseline (speedup 1.0000x reference)
import jax
import jax.numpy as jnp
from jax.experimental import pallas as pl


def kernel(x, edge_index, W_in, b_in, W_out, b_out, W_root, b_root):
    raise NotImplementedError("write your pallas kernel here")


# speedup vs baseline: 3.8672x; 3.8672x over previous
import functools

import jax
import jax.numpy as jnp
from jax import lax
from jax.experimental import pallas as pl
from jax.experimental.pallas import tpu as pltpu
from jax.experimental.pallas import tpu_sc as plsc

N = 10000
D = 256
DH = 128
E = 160000
ALPHA = 0.5

NC = 2
NS = 16
K = 128
NCH = 79
EPT = K * NCH
E_PAD = NS * EPT

N_PAD = 10240
RPT = N_PAD // NS
ZCH = 64

N_TC = 10240
NB = 512


def _sc_aggregate(x2, srcs, dsts):
    mesh = plsc.VectorSubcoreMesh(
        core_axis_name="c", subcore_axis_name="s", num_cores=NC, num_subcores=NS)

    @functools.partial(
        pl.kernel,
        out_type=(
            jax.ShapeDtypeStruct((NC * N_PAD, DH), jnp.float32),
            jax.ShapeDtypeStruct((NS, N_PAD), jnp.float32),
        ),
        mesh=mesh,
        compiler_params=pltpu.CompilerParams(
            needs_layout_passes=False, use_tc_tiling_on_sc=False),
        scratch_types=[
            pltpu.VMEM((K,), jnp.int32),
            pltpu.VMEM((K,), jnp.int32),
            pltpu.VMEM((K, DH), jnp.float32),
            pltpu.VMEM((N_PAD,), jnp.float32),
            pltpu.VMEM((ZCH, DH), jnp.float32),
            pltpu.VMEM_SHARED((N_PAD, DH), jnp.float32),
            pltpu.SemaphoreType.DMA,
        ],
    )
    def k(x2_hbm, srcs_hbm, dsts_hbm, sum_hbm, cnt_hbm,
          src_v, dst_v, rows_v, cnt_v, z_v, acc, sem):
        c = lax.axis_index("c")
        s = lax.axis_index("s")
        zero16 = jnp.zeros((16,), jnp.float32)
        ones16 = jnp.ones((16,), jnp.float32)

        def zrow(i, _):
            for j in range(DH // 16):
                z_v[i, pl.ds(j * 16, 16)] = zero16
            return 0
        lax.fori_loop(0, ZCH, zrow, 0)

        def zcnt(i, _):
            cnt_v[pl.ds(i * 16, 16)] = zero16
            return 0
        lax.fori_loop(0, N_PAD // 16, zcnt, 0)

        def zacc(i, _):
            pltpu.sync_copy(z_v, acc.at[pl.ds(s * RPT + i * ZCH, ZCH)])
            return 0
        lax.fori_loop(0, RPT // ZCH, zacc, 0)

        plsc.subcore_barrier()

        def body(i, _):
            base = s * EPT + i * K
            pltpu.sync_copy(srcs_hbm.at[c, pl.ds(base, K)], src_v)
            pltpu.sync_copy(dsts_hbm.at[pl.ds(base, K)], dst_v)
            pltpu.async_copy(x2_hbm.at[src_v], rows_v, sem).wait()
            pltpu.sync_copy(rows_v, acc.at[dst_v], add=True)

            @pl.when(c == 0)
            def _():
                for j in range(K // 16):
                    idx16 = dst_v[pl.ds(j * 16, 16)]
                    plsc.addupdate_scatter(cnt_v, [idx16], ones16)
            return 0
        lax.fori_loop(0, NCH, body, 0)

        plsc.subcore_barrier()

        pltpu.sync_copy(acc.at[pl.ds(s * RPT, RPT)],
                        sum_hbm.at[pl.ds(c * N_PAD + s * RPT, RPT)])

        @pl.when(c == 0)
        def _():
            pltpu.sync_copy(cnt_v, cnt_hbm.at[s])

    return k(x2, srcs, dsts)


def _tc_body(x_ref, s0_ref, s1_ref, cnt_ref, wi_ref, wo_ref, wr_ref,
             bi_ref, bo_ref, br_ref, o_ref):
    cnt = jnp.sum(cnt_ref[...], axis=0)
    inv = 1.0 / jnp.maximum(cnt, 1.0)
    m = jnp.concatenate([s0_ref[...], s1_ref[...]], axis=1) * inv[:, None]
    wc = ALPHA * wo_ref[...] + (1.0 - ALPHA) * wi_ref[...]
    bias = ALPHA * bo_ref[...] + (1.0 - ALPHA) * bi_ref[...] + br_ref[...]
    dot = functools.partial(jnp.dot, preferred_element_type=jnp.float32,
                            precision=lax.Precision.HIGHEST)
    o_ref[...] = dot(m, wc) + dot(x_ref[...], wr_ref[...]) + bias


def _tc_combine(x_pad, sum_flat, cnt_parts, W_in, W_out, W_root,
                b_in, b_out, b_root):
    grid = (N_TC // NB,)
    full = lambda i: (0, 0)
    return pl.pallas_call(
        _tc_body,
        grid=grid,
        in_specs=[
            pl.BlockSpec((NB, D), lambda i: (i, 0)),
            pl.BlockSpec((NB, DH), lambda i: (i, 0)),
            pl.BlockSpec((NB, DH), lambda i: (i + N_PAD // NB, 0)),
            pl.BlockSpec((NS, NB), lambda i: (0, i)),
            pl.BlockSpec((D, D), full),
            pl.BlockSpec((D, D), full),
            pl.BlockSpec((D, D), full),
            pl.BlockSpec((1, D), full),
            pl.BlockSpec((1, D), full),
            pl.BlockSpec((1, D), full),
        ],
        out_specs=pl.BlockSpec((NB, D), lambda i: (i, 0)),
        out_shape=jax.ShapeDtypeStruct((N_TC, D), jnp.float32),
    )(x_pad, sum_flat, sum_flat, cnt_parts, W_in, W_out, W_root,
      b_in.reshape(1, D), b_out.reshape(1, D), b_root.reshape(1, D))


def kernel(x, edge_index, W_in, b_in, W_out, b_out, W_root, b_root):
    src = edge_index[0]
    dst = edge_index[1]
    pad = E_PAD - E
    src_p = jnp.concatenate([src, jnp.zeros((pad,), jnp.int32)])
    dst_p = jnp.concatenate([dst, jnp.full((pad,), N, jnp.int32)])
    srcs = jnp.stack([src_p, src_p + N])
    x2 = jnp.concatenate([x[:, :DH], x[:, DH:]], axis=0)
    sum_flat, cnt_parts = _sc_aggregate(x2, srcs, dst_p)
    x_pad = jnp.pad(x, ((0, N_TC - N), (0, 0)))
    out_pad = _tc_combine(x_pad, sum_flat, cnt_parts, W_in, W_out, W_root,
                          b_in, b_out, b_root)
    return out_pad[:N]


# speedup vs baseline: 4.4501x; 1.1507x over previous
import functools

import jax
import jax.numpy as jnp
from jax import lax
from jax.experimental import pallas as pl
from jax.experimental.pallas import tpu as pltpu
from jax.experimental.pallas import tpu_sc as plsc

N = 10000
D = 256
DH = 128
E = 160000
ALPHA = 0.5

NC = 2
NS = 16
K = 128
NCH = 79
EPT = K * NCH
E_PAD = NS * EPT

N_PAD = 10240
RPT = N_PAD // NS

N_TC = 10240
NB = 512


def _sc_aggregate(x2, srcs3, dsts3):
    mesh = plsc.VectorSubcoreMesh(
        core_axis_name="c", subcore_axis_name="s", num_cores=NC, num_subcores=NS)

    @functools.partial(
        pl.kernel,
        out_type=(
            jax.ShapeDtypeStruct((NC * N_PAD, DH), jnp.float32),
            jax.ShapeDtypeStruct((NS, N_PAD), jnp.float32),
        ),
        mesh=mesh,
        compiler_params=pltpu.CompilerParams(
            needs_layout_passes=False, use_tc_tiling_on_sc=False),
        scratch_types=[
            pltpu.VMEM((NCH, K), jnp.int32),
            pltpu.VMEM((NCH, K), jnp.int32),
            pltpu.VMEM((K,), jnp.int32),
            pltpu.VMEM((K,), jnp.int32),
            pltpu.VMEM((K, DH), jnp.float32),
            pltpu.VMEM((N_PAD,), jnp.float32),
            pltpu.VMEM_SHARED((N_PAD, DH), jnp.float32),
            pltpu.SemaphoreType.DMA,
        ],
    )
    def k(x2_hbm, srcs_hbm, dsts_hbm, sum_hbm, cnt_hbm,
          src_all, dst_all, src_v, dst_v, rows_v, cnt_v, acc, sem):
        c = lax.axis_index("c")
        s = lax.axis_index("s")
        zero16 = jnp.zeros((16,), jnp.float32)
        ones16 = jnp.ones((16,), jnp.float32)

        pltpu.sync_copy(srcs_hbm.at[c, s], src_all)
        pltpu.sync_copy(dsts_hbm.at[s], dst_all)

        def zrow(i, _):
            for j in range(DH // 16):
                rows_v[i, pl.ds(j * 16, 16)] = zero16
            return 0
        lax.fori_loop(0, K, zrow, 0)

        def zcnt(i, _):
            cnt_v[pl.ds(i * 16, 16)] = zero16
            return 0
        lax.fori_loop(0, N_PAD // 16, zcnt, 0)

        def zacc(i, _):
            pltpu.sync_copy(rows_v, acc.at[pl.ds(s * RPT + i * K, K)])
            return 0
        lax.fori_loop(0, RPT // K, zacc, 0)

        plsc.subcore_barrier()

        def body(i, _):
            for j in range(K // 16):
                s16 = src_all[i, pl.ds(j * 16, 16)]
                src_v[pl.ds(j * 16, 16)] = s16
                d16 = dst_all[i, pl.ds(j * 16, 16)]
                dst_v[pl.ds(j * 16, 16)] = d16

                @pl.when(c == 0)
                def _():
                    plsc.addupdate_scatter(cnt_v, [d16], ones16)
            pltpu.async_copy(x2_hbm.at[src_v], rows_v, sem).wait()
            pltpu.sync_copy(rows_v, acc.at[dst_v], add=True)
            return 0
        lax.fori_loop(0, NCH, body, 0)

        plsc.subcore_barrier()

        pltpu.sync_copy(acc.at[pl.ds(s * RPT, RPT)],
                        sum_hbm.at[pl.ds(c * N_PAD + s * RPT, RPT)])

        @pl.when(c == 0)
        def _():
            pltpu.sync_copy(cnt_v, cnt_hbm.at[s])

    return k(x2, srcs3, dsts3)


def _tc_body(x_ref, s0_ref, s1_ref, cnt_ref, wi_ref, wo_ref, wr_ref,
             bi_ref, bo_ref, br_ref, o_ref):
    cnt = jnp.sum(cnt_ref[...], axis=0)
    inv = 1.0 / jnp.maximum(cnt, 1.0)
    m = jnp.concatenate([s0_ref[...], s1_ref[...]], axis=1) * inv[:, None]
    wc = ALPHA * wo_ref[...] + (1.0 - ALPHA) * wi_ref[...]
    bias = ALPHA * bo_ref[...] + (1.0 - ALPHA) * bi_ref[...] + br_ref[...]
    dot = functools.partial(jnp.dot, preferred_element_type=jnp.float32,
                            precision=lax.Precision.HIGHEST)
    o_ref[...] = dot(m, wc) + dot(x_ref[...], wr_ref[...]) + bias


def _tc_combine(x_pad, sum_flat, cnt_parts, W_in, W_out, W_root,
                b_in, b_out, b_root):
    grid = (N_TC // NB,)
    full = lambda i: (0, 0)
    return pl.pallas_call(
        _tc_body,
        grid=grid,
        in_specs=[
            pl.BlockSpec((NB, D), lambda i: (i, 0)),
            pl.BlockSpec((NB, DH), lambda i: (i, 0)),
            pl.BlockSpec((NB, DH), lambda i: (i + N_PAD // NB, 0)),
            pl.BlockSpec((NS, NB), lambda i: (0, i)),
            pl.BlockSpec((D, D), full),
            pl.BlockSpec((D, D), full),
            pl.BlockSpec((D, D), full),
            pl.BlockSpec((1, D), full),
            pl.BlockSpec((1, D), full),
            pl.BlockSpec((1, D), full),
        ],
        out_specs=pl.BlockSpec((NB, D), lambda i: (i, 0)),
        out_shape=jax.ShapeDtypeStruct((N_TC, D), jnp.float32),
    )(x_pad, sum_flat, sum_flat, cnt_parts, W_in, W_out, W_root,
      b_in.reshape(1, D), b_out.reshape(1, D), b_root.reshape(1, D))


def kernel(x, edge_index, W_in, b_in, W_out, b_out, W_root, b_root):
    src = edge_index[0]
    dst = edge_index[1]
    pad = E_PAD - E
    src_p = jnp.concatenate([src, jnp.zeros((pad,), jnp.int32)])
    dst_p = jnp.concatenate([dst, jnp.full((pad,), N, jnp.int32)])
    srcs3 = jnp.stack([src_p, src_p + N]).reshape(NC, NS, NCH, K)
    dsts3 = dst_p.reshape(NS, NCH, K)
    x2 = jnp.concatenate([x[:, :DH], x[:, DH:]], axis=0)
    sum_flat, cnt_parts = _sc_aggregate(x2, srcs3, dsts3)
    x_pad = jnp.pad(x, ((0, N_TC - N), (0, 0)))
    out_pad = _tc_combine(x_pad, sum_flat, cnt_parts, W_in, W_out, W_root,
                          b_in, b_out, b_root)
    return out_pad[:N]


# speedup vs baseline: 4.5457x; 1.0215x over previous
import functools

import jax
import jax.numpy as jnp
from jax import lax
from jax.experimental import pallas as pl
from jax.experimental.pallas import tpu as pltpu
from jax.experimental.pallas import tpu_sc as plsc

N = 10000
D = 256
DH = 128
E = 160000
ALPHA = 0.5

NC = 2
NS = 16
K = 64
NCH = 158
EPT = K * NCH
E_PAD = NS * EPT

N_PAD = 10240
RPT = N_PAD // NS

N_TC = 10240
NB = 512


def _sc_aggregate(x2, srcs3, dsts3):
    mesh = plsc.VectorSubcoreMesh(
        core_axis_name="c", subcore_axis_name="s", num_cores=NC, num_subcores=NS)

    @functools.partial(
        pl.kernel,
        out_type=(
            jax.ShapeDtypeStruct((NC * N_PAD, DH), jnp.float32),
            jax.ShapeDtypeStruct((NS, N_PAD), jnp.float32),
        ),
        mesh=mesh,
        compiler_params=pltpu.CompilerParams(
            needs_layout_passes=False, use_tc_tiling_on_sc=False),
        scratch_types=[
            pltpu.VMEM((NCH, K), jnp.int32),
            pltpu.VMEM((NCH, K), jnp.int32),
            [pltpu.VMEM((K,), jnp.int32) for _ in range(2)],
            [pltpu.VMEM((K,), jnp.int32) for _ in range(2)],
            [pltpu.VMEM((K, DH), jnp.float32) for _ in range(2)],
            pltpu.VMEM((N_PAD,), jnp.float32),
            pltpu.VMEM_SHARED((N_PAD, DH), jnp.float32),
            [pltpu.SemaphoreType.DMA for _ in range(4)],
        ],
    )
    def k(x2_hbm, srcs_hbm, dsts_hbm, sum_hbm, cnt_hbm,
          src_all, dst_all, src_v, dst_v, rows, cnt_v, acc, sems):
        c = lax.axis_index("c")
        s = lax.axis_index("s")
        zero16 = jnp.zeros((16,), jnp.float32)
        ones16 = jnp.ones((16,), jnp.float32)

        pltpu.sync_copy(srcs_hbm.at[c, s], src_all)
        pltpu.sync_copy(dsts_hbm.at[s], dst_all)

        def zrow(i, _):
            for j in range(DH // 16):
                rows[0][i, pl.ds(j * 16, 16)] = zero16
            return 0
        lax.fori_loop(0, K, zrow, 0)

        def zcnt(i, _):
            cnt_v[pl.ds(i * 16, 16)] = zero16
            return 0
        lax.fori_loop(0, N_PAD // 16, zcnt, 0)

        def zacc(i, _):
            pltpu.sync_copy(rows[0], acc.at[pl.ds(s * RPT + i * K, K)])
            return 0
        lax.fori_loop(0, RPT // K, zacc, 0)

        plsc.subcore_barrier()

        def body(i, _):
            for b in range(2):
                ch = 2 * i + b
                for j in range(K // 16):
                    s16 = src_all[ch, pl.ds(j * 16, 16)]
                    src_v[b][pl.ds(j * 16, 16)] = s16
                    d16 = dst_all[ch, pl.ds(j * 16, 16)]
                    dst_v[b][pl.ds(j * 16, 16)] = d16

                    @pl.when(c == 0)
                    def _():
                        plsc.addupdate_scatter(cnt_v, [d16], ones16)
            g0 = pltpu.async_copy(x2_hbm.at[src_v[0]], rows[0], sems[0])
            g1 = pltpu.async_copy(x2_hbm.at[src_v[1]], rows[1], sems[1])
            g0.wait()
            s0 = pltpu.async_copy(rows[0], acc.at[dst_v[0]], sems[2],
                                  add=True)
            g1.wait()
            s1 = pltpu.async_copy(rows[1], acc.at[dst_v[1]], sems[3],
                                  add=True)
            s0.wait()
            s1.wait()
            return 0
        lax.fori_loop(0, NCH // 2, body, 0)

        plsc.subcore_barrier()

        pltpu.sync_copy(acc.at[pl.ds(s * RPT, RPT)],
                        sum_hbm.at[pl.ds(c * N_PAD + s * RPT, RPT)])

        @pl.when(c == 0)
        def _():
            pltpu.sync_copy(cnt_v, cnt_hbm.at[s])

    return k(x2, srcs3, dsts3)


def _tc_body(x_ref, s0_ref, s1_ref, cnt_ref, wi_ref, wo_ref, wr_ref,
             bi_ref, bo_ref, br_ref, o_ref):
    cnt = jnp.sum(cnt_ref[...], axis=0)
    inv = 1.0 / jnp.maximum(cnt, 1.0)
    m = jnp.concatenate([s0_ref[...], s1_ref[...]], axis=1) * inv[:, None]
    wc = ALPHA * wo_ref[...] + (1.0 - ALPHA) * wi_ref[...]
    bias = ALPHA * bo_ref[...] + (1.0 - ALPHA) * bi_ref[...] + br_ref[...]
    dot = functools.partial(jnp.dot, preferred_element_type=jnp.float32,
                            precision=lax.Precision.HIGHEST)
    o_ref[...] = dot(m, wc) + dot(x_ref[...], wr_ref[...]) + bias


def _tc_combine(x_pad, sum_flat, cnt_parts, W_in, W_out, W_root,
                b_in, b_out, b_root):
    grid = (N_TC // NB,)
    full = lambda i: (0, 0)
    return pl.pallas_call(
        _tc_body,
        grid=grid,
        in_specs=[
            pl.BlockSpec((NB, D), lambda i: (i, 0)),
            pl.BlockSpec((NB, DH), lambda i: (i, 0)),
            pl.BlockSpec((NB, DH), lambda i: (i + N_PAD // NB, 0)),
            pl.BlockSpec((NS, NB), lambda i: (0, i)),
            pl.BlockSpec((D, D), full),
            pl.BlockSpec((D, D), full),
            pl.BlockSpec((D, D), full),
            pl.BlockSpec((1, D), full),
            pl.BlockSpec((1, D), full),
            pl.BlockSpec((1, D), full),
        ],
        out_specs=pl.BlockSpec((NB, D), lambda i: (i, 0)),
        out_shape=jax.ShapeDtypeStruct((N_TC, D), jnp.float32),
    )(x_pad, sum_flat, sum_flat, cnt_parts, W_in, W_out, W_root,
      b_in.reshape(1, D), b_out.reshape(1, D), b_root.reshape(1, D))


def kernel(x, edge_index, W_in, b_in, W_out, b_out, W_root, b_root):
    src = edge_index[0]
    dst = edge_index[1]
    pad = E_PAD - E
    src_p = jnp.concatenate([src, jnp.zeros((pad,), jnp.int32)])
    dst_p = jnp.concatenate([dst, jnp.full((pad,), N, jnp.int32)])
    srcs3 = jnp.stack([src_p, src_p + N]).reshape(NC, NS, NCH, K)
    dsts3 = dst_p.reshape(NS, NCH, K)
    x2 = jnp.concatenate([x[:, :DH], x[:, DH:]], axis=0)
    sum_flat, cnt_parts = _sc_aggregate(x2, srcs3, dsts3)
    x_pad = jnp.pad(x, ((0, N_TC - N), (0, 0)))
    out_pad = _tc_combine(x_pad, sum_flat, cnt_parts, W_in, W_out, W_root,
                          b_in, b_out, b_root)
    return out_pad[:N]


# speedup vs baseline: 4.6651x; 1.0263x over previous
import functools

import jax
import jax.numpy as jnp
from jax import lax
from jax.experimental import pallas as pl
from jax.experimental.pallas import tpu as pltpu
from jax.experimental.pallas import tpu_sc as plsc

N = 10000
D = 256
DH = 128
E = 160000
ALPHA = 0.5

NC = 2
NS = 16
K = 64
NCH = 158
EPT = K * NCH
E_PAD = NS * EPT

N_PAD = 10240
RPT = N_PAD // NS


NB = 400


def _sc_aggregate(x2, srcs4, dsts3):
    mesh = plsc.VectorSubcoreMesh(
        core_axis_name="c", subcore_axis_name="s", num_cores=NC, num_subcores=NS)

    @functools.partial(
        pl.kernel,
        out_type=(
            jax.ShapeDtypeStruct((NC, N_PAD, DH), jnp.float32),
            jax.ShapeDtypeStruct((NS, N_PAD), jnp.float32),
        ),
        mesh=mesh,
        compiler_params=pltpu.CompilerParams(
            needs_layout_passes=False, use_tc_tiling_on_sc=False),
        scratch_types=[
            pltpu.VMEM((NCH, K), jnp.int32),
            pltpu.VMEM((NCH, K), jnp.int32),
            [pltpu.VMEM((K,), jnp.int32) for _ in range(2)],
            [pltpu.VMEM((K,), jnp.int32) for _ in range(2)],
            [pltpu.VMEM((K, DH), jnp.float32) for _ in range(2)],
            pltpu.VMEM((N_PAD,), jnp.float32),
            pltpu.VMEM_SHARED((N_PAD, DH), jnp.float32),
            [pltpu.SemaphoreType.DMA for _ in range(4)],
        ],
    )
    def k(x2_hbm, srcs_hbm, dsts_hbm, sum_hbm, cnt_hbm,
          src_all, dst_all, src_v, dst_v, rows, cnt_v, acc, sems):
        c = lax.axis_index("c")
        s = lax.axis_index("s")
        zero16 = jnp.zeros((16,), jnp.float32)
        ones16 = jnp.ones((16,), jnp.float32)

        pltpu.sync_copy(srcs_hbm.at[c, s], src_all)
        pltpu.sync_copy(dsts_hbm.at[s], dst_all)

        def zrow(i, _):
            for j in range(DH // 16):
                rows[0][i, pl.ds(j * 16, 16)] = zero16
            return 0
        lax.fori_loop(0, K, zrow, 0)

        def zcnt(i, _):
            cnt_v[pl.ds(i * 16, 16)] = zero16
            return 0
        lax.fori_loop(0, N_PAD // 16, zcnt, 0)

        def zacc(i, _):
            pltpu.sync_copy(rows[0], acc.at[pl.ds(s * RPT + i * K, K)])
            return 0
        lax.fori_loop(0, RPT // K, zacc, 0)

        plsc.subcore_barrier()

        def body(i, _):
            for b in range(2):
                ch = 2 * i + b
                for j in range(K // 16):
                    s16 = src_all[ch, pl.ds(j * 16, 16)]
                    src_v[b][pl.ds(j * 16, 16)] = s16
                    d16 = dst_all[ch, pl.ds(j * 16, 16)]
                    dst_v[b][pl.ds(j * 16, 16)] = d16

                    @pl.when(c == 0)
                    def _():
                        plsc.addupdate_scatter(cnt_v, [d16], ones16)
            g0 = pltpu.async_copy(x2_hbm.at[src_v[0]], rows[0], sems[0])
            g1 = pltpu.async_copy(x2_hbm.at[src_v[1]], rows[1], sems[1])
            g0.wait()
            s0 = pltpu.async_copy(rows[0], acc.at[dst_v[0]], sems[2],
                                  add=True)
            g1.wait()
            s1 = pltpu.async_copy(rows[1], acc.at[dst_v[1]], sems[3],
                                  add=True)
            s0.wait()
            s1.wait()
            return 0
        lax.fori_loop(0, NCH // 2, body, 0)

        plsc.subcore_barrier()

        pltpu.sync_copy(acc.at[pl.ds(s * RPT, RPT)],
                        sum_hbm.at[c, pl.ds(s * RPT, RPT)])

        @pl.when(c == 0)
        def _():
            pltpu.sync_copy(cnt_v, cnt_hbm.at[s])

    return k(x2, srcs4, dsts3)


def _tc_body(x_ref, s0_ref, s1_ref, cnt_ref, wi_ref, wo_ref, wr_ref,
             bi_ref, bo_ref, br_ref, o_ref):
    inv = 1.0 / jnp.maximum(cnt_ref[...], 1.0)
    m = jnp.concatenate([s0_ref[0], s1_ref[0]], axis=1) * inv
    wc = ALPHA * wo_ref[...] + (1.0 - ALPHA) * wi_ref[...]
    bias = ALPHA * bo_ref[...] + (1.0 - ALPHA) * bi_ref[...] + br_ref[...]
    dot = functools.partial(jnp.dot, preferred_element_type=jnp.float32)
    o_ref[...] = dot(m, wc) + dot(x_ref[...], wr_ref[...]) + bias


def _tc_combine(x, sum3, cnt_col, W_in, W_out, W_root,
                b_in, b_out, b_root):
    grid = (N // NB,)
    full = lambda i: (0, 0)
    return pl.pallas_call(
        _tc_body,
        grid=grid,
        in_specs=[
            pl.BlockSpec((NB, D), lambda i: (i, 0)),
            pl.BlockSpec((1, NB, DH), lambda i: (0, i, 0)),
            pl.BlockSpec((1, NB, DH), lambda i: (1, i, 0)),
            pl.BlockSpec((NB, 1), lambda i: (i, 0)),
            pl.BlockSpec((D, D), full),
            pl.BlockSpec((D, D), full),
            pl.BlockSpec((D, D), full),
            pl.BlockSpec((1, D), full),
            pl.BlockSpec((1, D), full),
            pl.BlockSpec((1, D), full),
        ],
        out_specs=pl.BlockSpec((NB, D), lambda i: (i, 0)),
        out_shape=jax.ShapeDtypeStruct((N, D), jnp.float32),
    )(x, sum3, sum3, cnt_col, W_in, W_out, W_root,
      b_in.reshape(1, D), b_out.reshape(1, D), b_root.reshape(1, D))


def kernel(x, edge_index, W_in, b_in, W_out, b_out, W_root, b_root):
    src = edge_index[0]
    dst = edge_index[1]
    pad = E_PAD - E
    src_p = jnp.concatenate([src, jnp.zeros((pad,), jnp.int32)])
    dst_p = jnp.concatenate([dst, jnp.full((pad,), N, jnp.int32)])
    srcs4 = jnp.stack([src_p, src_p + N]).reshape(NC, NS, NCH, K)
    dsts3 = dst_p.reshape(NS, NCH, K)
    x2 = jnp.concatenate([x[:, :DH], x[:, DH:]], axis=0)
    sum3, cnt_parts = _sc_aggregate(x2, srcs4, dsts3)
    cnt_col = jnp.sum(cnt_parts, axis=0)[:N].reshape(N, 1)
    return _tc_combine(x, sum3, cnt_col, W_in, W_out, W_root,
                       b_in, b_out, b_root)


# speedup vs baseline: 5.5391x; 1.1873x over previous
import functools

import jax
import jax.numpy as jnp
from jax import lax
from jax.experimental import pallas as pl
from jax.experimental.pallas import tpu as pltpu
from jax.experimental.pallas import tpu_sc as plsc

N = 10000
D = 256
DH = 128
E = 160000
ALPHA = 0.5

NC = 2
NS = 16
K = 64
NCH = 158
EPT = K * NCH
E_PAD = NS * EPT

N_PAD = 10240
RPT = N_PAD // NS


NB = 400


def _sc_aggregate(x2, srcs4, dsts3):
    mesh = plsc.VectorSubcoreMesh(
        core_axis_name="c", subcore_axis_name="s", num_cores=NC, num_subcores=NS)

    @functools.partial(
        pl.kernel,
        out_type=(
            jax.ShapeDtypeStruct((NC, N_PAD, DH), jnp.bfloat16),
            jax.ShapeDtypeStruct((NS, N_PAD), jnp.float32),
        ),
        mesh=mesh,
        compiler_params=pltpu.CompilerParams(
            needs_layout_passes=False, use_tc_tiling_on_sc=False),
        scratch_types=[
            pltpu.VMEM((NCH, K), jnp.int32),
            pltpu.VMEM((NCH, K), jnp.int32),
            [pltpu.VMEM((K,), jnp.int32) for _ in range(2)],
            [pltpu.VMEM((K,), jnp.int32) for _ in range(2)],
            [pltpu.VMEM((K, DH), jnp.bfloat16) for _ in range(2)],
            pltpu.VMEM((N_PAD,), jnp.float32),
            pltpu.VMEM_SHARED((N_PAD, DH), jnp.bfloat16),
            [pltpu.SemaphoreType.DMA for _ in range(4)],
        ],
    )
    def k(x2_hbm, srcs_hbm, dsts_hbm, sum_hbm, cnt_hbm,
          src_all, dst_all, src_v, dst_v, rows, cnt_v, acc, sems):
        c = lax.axis_index("c")
        s = lax.axis_index("s")
        zero16 = jnp.zeros((16,), jnp.float32)
        ones16 = jnp.ones((16,), jnp.float32)

        pltpu.sync_copy(srcs_hbm.at[c, s], src_all)
        pltpu.sync_copy(dsts_hbm.at[s], dst_all)

        zero32b = jnp.zeros((32,), jnp.bfloat16)

        def zrow(i, _):
            for j in range(DH // 32):
                rows[0][i, pl.ds(j * 32, 32)] = zero32b
            return 0
        lax.fori_loop(0, K, zrow, 0)

        def zcnt(i, _):
            cnt_v[pl.ds(i * 16, 16)] = zero16
            return 0
        lax.fori_loop(0, N_PAD // 16, zcnt, 0)

        def zacc(i, _):
            pltpu.sync_copy(rows[0], acc.at[pl.ds(s * RPT + i * K, K)])
            return 0
        lax.fori_loop(0, RPT // K, zacc, 0)

        plsc.subcore_barrier()

        def body(i, _):
            for b in range(2):
                ch = 2 * i + b
                for j in range(K // 16):
                    s16 = src_all[ch, pl.ds(j * 16, 16)]
                    src_v[b][pl.ds(j * 16, 16)] = s16
                    d16 = dst_all[ch, pl.ds(j * 16, 16)]
                    dst_v[b][pl.ds(j * 16, 16)] = d16

                    @pl.when(c == 0)
                    def _():
                        plsc.addupdate_scatter(cnt_v, [d16], ones16)
            g0 = pltpu.async_copy(x2_hbm.at[src_v[0]], rows[0], sems[0])
            g1 = pltpu.async_copy(x2_hbm.at[src_v[1]], rows[1], sems[1])
            g0.wait()
            s0 = pltpu.async_copy(rows[0], acc.at[dst_v[0]], sems[2],
                                  add=True)
            g1.wait()
            s1 = pltpu.async_copy(rows[1], acc.at[dst_v[1]], sems[3],
                                  add=True)
            s0.wait()
            s1.wait()
            return 0
        lax.fori_loop(0, NCH // 2, body, 0)

        plsc.subcore_barrier()

        pltpu.sync_copy(acc.at[pl.ds(s * RPT, RPT)],
                        sum_hbm.at[c, pl.ds(s * RPT, RPT)])

        @pl.when(c == 0)
        def _():
            pltpu.sync_copy(cnt_v, cnt_hbm.at[s])

    return k(x2, srcs4, dsts3)


def _tc_body(x_ref, s0_ref, s1_ref, cnt_ref, wi_ref, wo_ref, wr_ref,
             bi_ref, bo_ref, br_ref, o_ref):
    inv = 1.0 / jnp.maximum(cnt_ref[...], 1.0)
    m = jnp.concatenate([s0_ref[0], s1_ref[0]],
                        axis=1).astype(jnp.float32) * inv
    wc = ALPHA * wo_ref[...] + (1.0 - ALPHA) * wi_ref[...]
    bias = ALPHA * bo_ref[...] + (1.0 - ALPHA) * bi_ref[...] + br_ref[...]
    dot = functools.partial(jnp.dot, preferred_element_type=jnp.float32)
    o_ref[...] = dot(m, wc) + dot(x_ref[...], wr_ref[...]) + bias


def _tc_combine(x, sum3, cnt_col, W_in, W_out, W_root,
                b_in, b_out, b_root):
    grid = (N // NB,)
    full = lambda i: (0, 0)
    return pl.pallas_call(
        _tc_body,
        grid=grid,
        in_specs=[
            pl.BlockSpec((NB, D), lambda i: (i, 0)),
            pl.BlockSpec((1, NB, DH), lambda i: (0, i, 0)),
            pl.BlockSpec((1, NB, DH), lambda i: (1, i, 0)),
            pl.BlockSpec((NB, 1), lambda i: (i, 0)),
            pl.BlockSpec((D, D), full),
            pl.BlockSpec((D, D), full),
            pl.BlockSpec((D, D), full),
            pl.BlockSpec((1, D), full),
            pl.BlockSpec((1, D), full),
            pl.BlockSpec((1, D), full),
        ],
        out_specs=pl.BlockSpec((NB, D), lambda i: (i, 0)),
        out_shape=jax.ShapeDtypeStruct((N, D), jnp.float32),
    )(x, sum3, sum3, cnt_col, W_in, W_out, W_root,
      b_in.reshape(1, D), b_out.reshape(1, D), b_root.reshape(1, D))


def kernel(x, edge_index, W_in, b_in, W_out, b_out, W_root, b_root):
    src = edge_index[0]
    dst = edge_index[1]
    pad = E_PAD - E
    src_p = jnp.concatenate([src, jnp.zeros((pad,), jnp.int32)])
    dst_p = jnp.concatenate([dst, jnp.full((pad,), N, jnp.int32)])
    srcs4 = jnp.stack([src_p, src_p + N]).reshape(NC, NS, NCH, K)
    dsts3 = dst_p.reshape(NS, NCH, K)
    xb = x.astype(jnp.bfloat16)
    x2 = jnp.concatenate([xb[:, :DH], xb[:, DH:]], axis=0)
    sum3, cnt_parts = _sc_aggregate(x2, srcs4, dsts3)
    cnt_col = jnp.sum(cnt_parts, axis=0)[:N].reshape(N, 1)
    return _tc_combine(x, sum3, cnt_col, W_in, W_out, W_root,
                       b_in, b_out, b_root)


# speedup vs baseline: 5.6139x; 1.0135x over previous
import functools

import jax
import jax.numpy as jnp
from jax import lax
from jax.experimental import pallas as pl
from jax.experimental.pallas import tpu as pltpu
from jax.experimental.pallas import tpu_sc as plsc

N = 10000
D = 256
DH = 128
E = 160000
ALPHA = 0.5

NC = 2
NS = 16
K = 64
NCH = 79
EPT = K * NCH
E_PAD = NC * NS * EPT

N_PAD = 10240
RPT = N_PAD // NS


NB = 400


def _sc_aggregate(xb, srcs4, dsts4):
    mesh = plsc.VectorSubcoreMesh(
        core_axis_name="c", subcore_axis_name="s", num_cores=NC, num_subcores=NS)

    @functools.partial(
        pl.kernel,
        out_type=(
            jax.ShapeDtypeStruct((NC, N_PAD, D), jnp.bfloat16),
            jax.ShapeDtypeStruct((NC, NS, N_PAD), jnp.float32),
        ),
        mesh=mesh,
        compiler_params=pltpu.CompilerParams(
            needs_layout_passes=False, use_tc_tiling_on_sc=False),
        scratch_types=[
            pltpu.VMEM((NCH, K), jnp.int32),
            pltpu.VMEM((NCH, K), jnp.int32),
            [pltpu.VMEM((K,), jnp.int32) for _ in range(2)],
            [pltpu.VMEM((K,), jnp.int32) for _ in range(2)],
            [pltpu.VMEM((K, D), jnp.bfloat16) for _ in range(2)],
            pltpu.VMEM((N_PAD,), jnp.float32),
            pltpu.VMEM_SHARED((N_PAD, D), jnp.bfloat16),
            [pltpu.SemaphoreType.DMA for _ in range(4)],
        ],
    )
    def k(xb_hbm, srcs_hbm, dsts_hbm, sum_hbm, cnt_hbm,
          src_all, dst_all, src_v, dst_v, rows, cnt_v, acc, sems):
        c = lax.axis_index("c")
        s = lax.axis_index("s")
        zero16 = jnp.zeros((16,), jnp.float32)
        ones16 = jnp.ones((16,), jnp.float32)

        pltpu.sync_copy(srcs_hbm.at[c, s], src_all)
        pltpu.sync_copy(dsts_hbm.at[c, s], dst_all)

        zero32b = jnp.zeros((32,), jnp.bfloat16)

        def zrow(i, _):
            for j in range(D // 32):
                rows[0][i, pl.ds(j * 32, 32)] = zero32b
            return 0
        lax.fori_loop(0, K, zrow, 0)

        def zcnt(i, _):
            cnt_v[pl.ds(i * 16, 16)] = zero16
            return 0
        lax.fori_loop(0, N_PAD // 16, zcnt, 0)

        def zacc(i, _):
            pltpu.sync_copy(rows[0], acc.at[pl.ds(s * RPT + i * K, K)])
            return 0
        lax.fori_loop(0, RPT // K, zacc, 0)

        plsc.subcore_barrier()

        def body(i, _):
            for b in range(2):
                ch = 2 * i + b
                for j in range(K // 16):
                    s16 = src_all[ch, pl.ds(j * 16, 16)]
                    src_v[b][pl.ds(j * 16, 16)] = s16
                    d16 = dst_all[ch, pl.ds(j * 16, 16)]
                    dst_v[b][pl.ds(j * 16, 16)] = d16

                    plsc.addupdate_scatter(cnt_v, [d16], ones16)
            g0 = pltpu.async_copy(xb_hbm.at[src_v[0]], rows[0], sems[0])
            g1 = pltpu.async_copy(xb_hbm.at[src_v[1]], rows[1], sems[1])
            g0.wait()
            s0 = pltpu.async_copy(rows[0], acc.at[dst_v[0]], sems[2],
                                  add=True)
            g1.wait()
            s1 = pltpu.async_copy(rows[1], acc.at[dst_v[1]], sems[3],
                                  add=True)
            s0.wait()
            s1.wait()
            return 0
        lax.fori_loop(0, NCH // 2, body, 0)

        if NCH % 2:
            ch = NCH - 1
            for j in range(K // 16):
                s16 = src_all[ch, pl.ds(j * 16, 16)]
                src_v[0][pl.ds(j * 16, 16)] = s16
                d16 = dst_all[ch, pl.ds(j * 16, 16)]
                dst_v[0][pl.ds(j * 16, 16)] = d16
                plsc.addupdate_scatter(cnt_v, [d16], ones16)
            pltpu.async_copy(xb_hbm.at[src_v[0]], rows[0], sems[0]).wait()
            pltpu.async_copy(rows[0], acc.at[dst_v[0]], sems[2],
                             add=True).wait()

        plsc.subcore_barrier()

        pltpu.sync_copy(acc.at[pl.ds(s * RPT, RPT)],
                        sum_hbm.at[c, pl.ds(s * RPT, RPT)])

        pltpu.sync_copy(cnt_v, cnt_hbm.at[c, s])

    return k(xb, srcs4, dsts4)


def _tc_body(x_ref, s0_ref, s1_ref, cnt_ref, wi_ref, wo_ref, wr_ref,
             bi_ref, bo_ref, br_ref, o_ref):
    inv = 1.0 / jnp.maximum(cnt_ref[...], 1.0)
    m = (s0_ref[0].astype(jnp.float32) +
         s1_ref[0].astype(jnp.float32)) * inv
    wc = ALPHA * wo_ref[...] + (1.0 - ALPHA) * wi_ref[...]
    bias = ALPHA * bo_ref[...] + (1.0 - ALPHA) * bi_ref[...] + br_ref[...]
    dot = functools.partial(jnp.dot, preferred_element_type=jnp.float32)
    o_ref[...] = dot(m, wc) + dot(x_ref[...], wr_ref[...]) + bias


def _tc_combine(x, sum3, cnt_col, W_in, W_out, W_root,
                b_in, b_out, b_root):
    grid = (N // NB,)
    full = lambda i: (0, 0)
    return pl.pallas_call(
        _tc_body,
        grid=grid,
        in_specs=[
            pl.BlockSpec((NB, D), lambda i: (i, 0)),
            pl.BlockSpec((1, NB, D), lambda i: (0, i, 0)),
            pl.BlockSpec((1, NB, D), lambda i: (1, i, 0)),
            pl.BlockSpec((NB, 1), lambda i: (i, 0)),
            pl.BlockSpec((D, D), full),
            pl.BlockSpec((D, D), full),
            pl.BlockSpec((D, D), full),
            pl.BlockSpec((1, D), full),
            pl.BlockSpec((1, D), full),
            pl.BlockSpec((1, D), full),
        ],
        out_specs=pl.BlockSpec((NB, D), lambda i: (i, 0)),
        out_shape=jax.ShapeDtypeStruct((N, D), jnp.float32),
    )(x, sum3, sum3, cnt_col, W_in, W_out, W_root,
      b_in.reshape(1, D), b_out.reshape(1, D), b_root.reshape(1, D))


def kernel(x, edge_index, W_in, b_in, W_out, b_out, W_root, b_root):
    src = edge_index[0]
    dst = edge_index[1]
    pad = E_PAD - E
    src_p = jnp.concatenate([src, jnp.zeros((pad,), jnp.int32)])
    dst_p = jnp.concatenate([dst, jnp.full((pad,), N, jnp.int32)])
    srcs4 = src_p.reshape(NC, NS, NCH, K)
    dsts4 = dst_p.reshape(NC, NS, NCH, K)
    xb = x.astype(jnp.bfloat16)
    sum3, cnt_parts = _sc_aggregate(xb, srcs4, dsts4)
    cnt_col = jnp.sum(cnt_parts, axis=(0, 1))[:N].reshape(N, 1)
    return _tc_combine(x, sum3, cnt_col, W_in, W_out, W_root,
                       b_in, b_out, b_root)
